# baseline (speedup 1.0000x reference)
import functools

import jax
import jax.numpy as jnp
from jax import lax
from jax.experimental import pallas as pl
from jax.experimental.pallas import tpu as pltpu
from jax.experimental.pallas import tpu_sc as plsc

R = 128
V = 100000
L = 16
NC = 2
NS = 16
NW = NC * NS
RPW = R // NW

CH = 10000
NCHR = V // CH
CV = CH // L
U = 5
INNER = CV // U

_NEG_INF = float("-inf")


def _gather16(x, idx):
    dnums = lax.GatherDimensionNumbers(
        offset_dims=(), collapsed_slice_dims=(0,), start_index_map=(0,))
    return lax.gather(
        x, idx[:, None], dnums, slice_sizes=(1,),
        mode=lax.GatherScatterMode.PROMISE_IN_BOUNDS)


def _scan_chunk(buf, it_base, ms, mis):
    def body(i, carry):
        ms, mis = carry
        it = jnp.full((L,), it_base + i, jnp.int32)
        base = i * (U * L)
        ms_new = []
        mis_new = []
        for j in range(U):
            v = buf[pl.ds(base + j * L, L)]
            cmp = v > ms[j]
            ms_new.append(jnp.where(cmp, v, ms[j]))
            mis_new.append(jnp.where(cmp, it, mis[j]))
        return tuple(ms_new), tuple(mis_new)

    return lax.fori_loop(0, INNER, body, (ms, mis))


def _finish_row(ms, mis):
    lane = lax.iota(jnp.int32, L)
    m = ms[0]
    mi = mis[0] * (U * L) + lane
    for j in range(1, U):
        idx_j = mis[j] * (U * L) + (j * L) + lane
        better = (ms[j] > m) | ((ms[j] == m) & (idx_j < mi))
        m = jnp.where(better, ms[j], m)
        mi = jnp.where(better, idx_j, mi)
    for shift in (8, 4, 2, 1):
        perm = (lane + shift) & (L - 1)
        mp = _gather16(m, perm)
        mip = _gather16(mi, perm)
        better = (mp > m) | ((mp == m) & (mip < mi))
        m = jnp.where(better, mp, m)
        mi = jnp.where(better, mip, mi)
    return mi


@functools.partial(
    pl.kernel,
    mesh=plsc.VectorSubcoreMesh(core_axis_name="c", subcore_axis_name="s"),
    compiler_params=pltpu.CompilerParams(use_tc_tiling_on_sc=False),
    out_type=jax.ShapeDtypeStruct((NW, L), jnp.int32),
    scratch_types=[
        pltpu.VMEM((CH,), jnp.float32),
        pltpu.VMEM((CH,), jnp.float32),
        pltpu.VMEM((L,), jnp.int32),
        pltpu.SemaphoreType.DMA,
        pltpu.SemaphoreType.DMA,
    ],
)
def _argmax_sc(logits_hbm, out_hbm, buf0, buf1, out_v, sem0, sem1):
    cid = lax.axis_index("c")
    sid = lax.axis_index("s")
    wid = sid * NC + cid
    base_row = wid * RPW
    bufs = (buf0, buf1)
    sems = (sem0, sem1)

    def start(row, c, b):
        pltpu.make_async_copy(
            logits_hbm.at[row, pl.ds(c * CH, CH)], bufs[b], sems[b]).start()

    def wait(b):
        pltpu.make_async_copy(
            logits_hbm.at[0, pl.ds(0, CH)], bufs[b], sems[b]).wait()

    lane = lax.iota(jnp.int32, L)
    res = jnp.zeros((L,), jnp.int32)
    start(base_row, 0, 0)
    for r in range(RPW):
        ms0 = tuple(jnp.full((L,), _NEG_INF, jnp.float32) for _ in range(U))
        mis0 = tuple(jnp.zeros((L,), jnp.int32) for _ in range(U))

        def grp(g, carry, r=r):
            ms, mis = carry
            for b in (0, 1):
                c = g * 2 + b
                nc = c + 1

                @pl.when(nc < NCHR)
                def _():
                    start(base_row + r, nc, 1 - b)

                if r < RPW - 1:
                    @pl.when(nc == NCHR)
                    def _():
                        start(base_row + r + 1, 0, 1 - b)

                wait(b)
                ms, mis = _scan_chunk(bufs[b], c * INNER, ms, mis)
            return ms, mis

        ms, mis = lax.fori_loop(0, NCHR // 2, grp, (ms0, mis0))
        idx = _finish_row(ms, mis)
        res = jnp.where(lane == r, idx, res)
    out_v[...] = res
    pltpu.sync_copy(out_v, out_hbm.at[wid])


def kernel(logits):
    out = _argmax_sc(logits)
    return out[:, :RPW].reshape(R)


# speedup vs baseline: 1.7518x; 1.7518x over previous
import functools

import jax
import jax.numpy as jnp
from jax import lax
from jax.experimental import pallas as pl
from jax.experimental.pallas import tpu as pltpu
from jax.experimental.pallas import tpu_sc as plsc

R = 128
V = 100000
L = 16
NC = 2
NS = 16
NW = NC * NS
G = 16
RPG = R // G

HALF = 50048
NCH = 17
CW = 2944
CV = CW // L
U = 8
INNER = CV // U

_GSTART = (V - HALF - (NCH - 1) * CW) // L

_NEG_INF = float("-inf")


def _gather16(x, idx):
    dnums = lax.GatherDimensionNumbers(
        offset_dims=(), collapsed_slice_dims=(0,), start_index_map=(0,))
    return lax.gather(
        x, idx[:, None], dnums, slice_sizes=(1,),
        mode=lax.GatherScatterMode.PROMISE_IN_BOUNDS)


def _scan_row(buf, j, col_base, m, gi):
    lane = lax.iota(jnp.int32, L)

    def body(i, carry):
        ms, mis = carry
        it = jnp.full((L,), i, jnp.int32)
        base = i * (U * L)
        ms_new = []
        mis_new = []
        for k in range(U):
            v = buf[j, pl.ds(base + k * L, L)]
            cmp = v > ms[k]
            ms_new.append(jnp.where(cmp, v, ms[k]))
            mis_new.append(jnp.where(cmp, it, mis[k]))
        return tuple(ms_new), tuple(mis_new)

    ms0 = tuple(jnp.full((L,), _NEG_INF, jnp.float32) for _ in range(U))
    mis0 = tuple(jnp.zeros((L,), jnp.int32) for _ in range(U))
    ms, mis = lax.fori_loop(0, INNER, body, (ms0, mis0))

    mseg = ms[0]
    iseg = mis[0] * (U * L) + lane
    for k in range(1, U):
        idx_k = mis[k] * (U * L) + (k * L) + lane
        better = (ms[k] > mseg) | ((ms[k] == mseg) & (idx_k < iseg))
        mseg = jnp.where(better, ms[k], mseg)
        iseg = jnp.where(better, idx_k, iseg)
    gseg = iseg + jnp.full((L,), col_base, jnp.int32)

    upd = mseg > m
    return jnp.where(upd, mseg, m), jnp.where(upd, gseg, gi)


def _butterfly(m, gi):
    lane = lax.iota(jnp.int32, L)
    for shift in (8, 4, 2, 1):
        perm = (lane + shift) & (L - 1)
        mp = _gather16(m, perm)
        gip = _gather16(gi, perm)
        better = (mp > m) | ((mp == m) & (gip < gi))
        m = jnp.where(better, mp, m)
        gi = jnp.where(better, gip, gi)
    return m, gi


@functools.partial(
    pl.kernel,
    mesh=plsc.VectorSubcoreMesh(core_axis_name="c", subcore_axis_name="s"),
    out_type=(
        jax.ShapeDtypeStruct((NW, L), jnp.float32),
        jax.ShapeDtypeStruct((NW, L), jnp.int32),
    ),
    scratch_types=[
        pltpu.VMEM((RPG, CW), jnp.float32),
        pltpu.VMEM((RPG, CW), jnp.float32),
        pltpu.VMEM((L,), jnp.float32),
        pltpu.VMEM((L,), jnp.int32),
        pltpu.SemaphoreType.DMA,
        pltpu.SemaphoreType.DMA,
    ],
)
def _argmax_sc(logits_hbm, vals_hbm, idxs_hbm, buf0, buf1, vout, iout,
               sem0, sem1):
    cid = lax.axis_index("c")
    sid = lax.axis_index("s")
    wid = sid * NC + cid
    grp = wid // 2
    half = wid % 2
    row0 = grp * RPG
    col0 = half * HALF
    bufs = (buf0, buf1)
    sems = (sem0, sem1)
    lane = lax.iota(jnp.int32, L)

    def start(c, b):
        pltpu.make_async_copy(
            logits_hbm.at[pl.ds(row0, RPG), pl.ds(col0 + c * CW, CW)],
            bufs[b], sems[b]).start()

    def wait(b):
        pltpu.make_async_copy(
            logits_hbm.at[pl.ds(0, RPG), pl.ds(0, CW)],
            bufs[b], sems[b]).wait()

    def mask_tail(buf):
        @pl.when((half == 1))
        def _():
            neg = jnp.full((L,), _NEG_INF, jnp.float32)
            for j in range(RPG):
                for v in range(_GSTART, CV):
                    buf[j, pl.ds(v * L, L)] = neg

    def scan_chunk(c, b, pairs):
        cb = col0 + c * CW
        new = []
        for j in range(RPG):
            m, gi = pairs[j]
            new.append(_scan_row(bufs[b], j, cb, m, gi))
        return new

    start(0, 0)
    start(1, 1)
    pairs = [(jnp.full((L,), _NEG_INF, jnp.float32),
              jnp.zeros((L,), jnp.int32)) for _ in range(RPG)]
    wait(0)
    pairs = scan_chunk(0, 0, pairs)

    def pair_body(g, carry):
        pairs = [(carry[2 * j], carry[2 * j + 1]) for j in range(RPG)]
        for p in (0, 1):
            c = 1 + 2 * g + p
            nxt = c + 1

            @pl.when(nxt < NCH)
            def _():
                start(nxt, p)

            wait(1 - p)
            if p == 1:
                @pl.when(c == NCH - 1)
                def _():
                    mask_tail(bufs[(NCH - 1) % 2])
            pairs = scan_chunk(c, 1 - p, pairs)
        return tuple(x for pr in pairs for x in pr)

    flat = lax.fori_loop(0, (NCH - 1) // 2, pair_body,
                         tuple(x for pr in pairs for x in pr))
    pairs = [(flat[2 * j], flat[2 * j + 1]) for j in range(RPG)]

    resv = jnp.full((L,), _NEG_INF, jnp.float32)
    resi = jnp.zeros((L,), jnp.int32)
    for j in range(RPG):
        m, gi = _butterfly(*pairs[j])
        resv = jnp.where(lane == j, m, resv)
        resi = jnp.where(lane == j, gi, resi)
    vout[...] = resv
    iout[...] = resi
    pltpu.sync_copy(vout, vals_hbm.at[wid])
    pltpu.sync_copy(iout, idxs_hbm.at[wid])


def kernel(logits):
    vals, idxs = _argmax_sc(logits)
    v = vals.reshape(G, 2, L)[:, :, :RPG]
    i = idxs.reshape(G, 2, L)[:, :, :RPG]
    pick = v[:, 1, :] > v[:, 0, :]
    out = jnp.where(pick, i[:, 1, :], i[:, 0, :])
    return out.reshape(R)


# speedup vs baseline: 3.6172x; 2.0649x over previous
import functools

import jax
import jax.numpy as jnp
from jax import lax
from jax.experimental import pallas as pl
from jax.experimental.pallas import tpu as pltpu
from jax.experimental.pallas import tpu_sc as plsc

R = 128
V = 100000
L = 16
NB = R // L
NC = 2
NS = 16
NW = NC * NS

SLAB = 3200
CPOS = 400
PU = 2

_NEG_INF = float("-inf")


@functools.partial(
    pl.kernel,
    mesh=plsc.VectorSubcoreMesh(core_axis_name="c", subcore_axis_name="s"),
    out_type=(
        jax.ShapeDtypeStruct((NW, R), jnp.float32),
        jax.ShapeDtypeStruct((NW, R), jnp.int32),
    ),
    scratch_types=[
        pltpu.VMEM((CPOS, R), jnp.float32),
        pltpu.VMEM((CPOS, R), jnp.float32),
        pltpu.VMEM((R,), jnp.float32),
        pltpu.VMEM((R,), jnp.int32),
        pltpu.SemaphoreType.DMA,
        pltpu.SemaphoreType.DMA,
    ],
)
def _argmax_sc(lt_hbm, vals_hbm, idxs_hbm, buf0, buf1, vout, iout,
               sem0, sem1):
    cid = lax.axis_index("c")
    sid = lax.axis_index("s")
    wid = sid * NC + cid
    off = wid * SLAB
    size = jnp.minimum(SLAB, V - off)
    nch = size // CPOS
    bufs = (buf0, buf1)
    sems = (sem0, sem1)

    def start(c, b):
        pltpu.make_async_copy(
            lt_hbm.at[pl.ds(off + c * CPOS, CPOS), :], bufs[b], sems[b]
        ).start()

    def wait(b):
        pltpu.make_async_copy(
            lt_hbm.at[pl.ds(0, CPOS), :], bufs[b], sems[b]).wait()

    def scan_chunk(c, b, carry):
        base = off + c * CPOS
        buf = bufs[b]

        def body(i, carry):
            ms, mis = carry
            ms, mis = list(ms), list(mis)
            for q in range(PU):
                p = i * PU + q
                it = jnp.full((L,), base + p, jnp.int32)
                for k in range(NB):
                    v = buf[p, pl.ds(k * L, L)]
                    cmp = v > ms[k]
                    ms[k] = jnp.where(cmp, v, ms[k])
                    mis[k] = jnp.where(cmp, it, mis[k])
            return tuple(ms), tuple(mis)

        return lax.fori_loop(0, CPOS // PU, body, carry)

    start(0, 0)
    ms0 = tuple(jnp.full((L,), _NEG_INF, jnp.float32) for _ in range(NB))
    mis0 = tuple(jnp.zeros((L,), jnp.int32) for _ in range(NB))

    def pair_body(g, flat):
        carry = (flat[:NB], flat[NB:])
        for p in (0, 1):
            c = 2 * g + p

            @pl.when(c + 1 < nch)
            def _():
                start(c + 1, 1 - p)

            wait(p)
            carry = scan_chunk(c, p, carry)
        return carry[0] + carry[1]

    flat = lax.fori_loop(0, nch // 2, pair_body, ms0 + mis0)
    for k in range(NB):
        vout[pl.ds(k * L, L)] = flat[k]
        iout[pl.ds(k * L, L)] = flat[NB + k]
    pltpu.sync_copy(vout, vals_hbm.at[wid])
    pltpu.sync_copy(iout, idxs_hbm.at[wid])


def kernel(logits):
    vals, idxs = _argmax_sc(logits.T)
    m = jnp.max(vals, axis=0)
    cand = jnp.where(vals == m[None, :], idxs, jnp.int32(V))
    return jnp.min(cand, axis=0)
